# Initial kernel scaffold; baseline (speedup 1.0000x reference)
import jax
import jax.numpy as jnp
from jax.experimental import pallas as pl


def kernel(tokens, cu_seqlens, table, W, b):
    raise NotImplementedError("write your pallas kernel here")


# speedup vs baseline: 1.3611x; 1.3611x over previous
import functools

import jax
import jax.numpy as jnp
from jax import lax
from jax.experimental import pallas as pl
from jax.experimental.pallas import tpu as pltpu
from jax.experimental.pallas import tpu_sc as plsc

TOTAL = 32768
D = 128
NC = 2
NS = 16
L = 16
NW = NC * NS
TPW = TOTAL // NW
CH = 128
NCH = TPW // CH

_mesh = plsc.VectorSubcoreMesh(core_axis_name="c", subcore_axis_name="s")


@functools.partial(
    pl.kernel,
    mesh=_mesh,
    out_type=jax.ShapeDtypeStruct((TOTAL,), jnp.float32),
    compiler_params=pltpu.CompilerParams(needs_layout_passes=False),
    scratch_types=[
        pltpu.VMEM((TPW,), jnp.int32),
        pltpu.VMEM((CH, D), jnp.float32),
        pltpu.VMEM((CH, D), jnp.float32),
        pltpu.VMEM((TPW,), jnp.float32),
        pltpu.VMEM((136,), jnp.float32),
        pltpu.SemaphoreType.DMA,
        pltpu.SemaphoreType.DMA,
    ],
)
def _sc_cost_kernel(tokens_hbm, table_hbm, wb_hbm, out_hbm,
                    idx_v, rows0, rows1, out_v, wb_v, sem0, sem1):
    wid = lax.axis_index("s") * NC + lax.axis_index("c")
    base = wid * TPW

    pltpu.sync_copy(wb_hbm, wb_v)
    pltpu.sync_copy(tokens_hbm.at[pl.ds(base, TPW)], idx_v)

    rows = [rows0, rows1]
    sems = [sem0, sem1]

    def start_gather(c):
        buf = c % 2
        return pltpu.async_copy(
            table_hbm.at[idx_v.at[pl.ds(c * CH, CH)]], rows[buf], sems[buf])

    lane = lax.broadcasted_iota(jnp.int32, (L,), 0)
    bias_vec = plsc.load_gather(wb_v, [jnp.full((L,), D, jnp.int32)])
    G = CH // L

    cp = start_gather(0)
    for c in range(NCH):
        nxt = start_gather(c + 1) if c + 1 < NCH else None
        cp.wait()
        buf = rows[c % 2]

        ridx = tuple(lane + g * L for g in range(G))
        accs0 = (bias_vec,) * G

        def body(d, carry):
            accs, cidx = carry
            wv = plsc.load_gather(wb_v, [cidx])
            new_accs = tuple(
                accs[g] + plsc.load_gather(buf, [ridx[g], cidx]) * wv
                for g in range(G))
            return new_accs, cidx + 1

        accs, _ = lax.fori_loop(
            0, D, body, (accs0, jnp.zeros((L,), jnp.int32)))
        for g in range(G):
            out_v[pl.ds(c * CH + g * L, L)] = accs[g]
        cp = nxt

    pltpu.sync_copy(out_v, out_hbm.at[pl.ds(base, TPW)])


def kernel(tokens, cu_seqlens, table, W, b):
    wb = jnp.concatenate([W[:, 0], b, jnp.zeros((7,), jnp.float32)])
    costs = _sc_cost_kernel(tokens, table, wb)
    return costs, cu_seqlens


# speedup vs baseline: 2.3330x; 1.7140x over previous
import functools

import jax
import jax.numpy as jnp
from jax import lax
from jax.experimental import pallas as pl
from jax.experimental.pallas import tpu as pltpu
from jax.experimental.pallas import tpu_sc as plsc

TOTAL = 32768
D = 128
NC = 2
NS = 16
L = 16
NW = NC * NS
TPW = TOTAL // NW
CH = 128
NCH = TPW // CH

_mesh = plsc.VectorSubcoreMesh(core_axis_name="c", subcore_axis_name="s")


@functools.partial(
    pl.kernel,
    mesh=_mesh,
    out_type=jax.ShapeDtypeStruct((TOTAL,), jnp.float32),
    compiler_params=pltpu.CompilerParams(needs_layout_passes=False),
    scratch_types=[
        pltpu.VMEM((TPW,), jnp.int32),
        pltpu.VMEM((CH, D), jnp.float32),
        pltpu.VMEM((CH, D), jnp.float32),
        pltpu.VMEM((TPW,), jnp.float32),
        pltpu.VMEM((136,), jnp.float32),
        pltpu.VMEM((D * L,), jnp.int32),
        pltpu.VMEM((D * L,), jnp.float32),
        pltpu.SemaphoreType.DMA,
        pltpu.SemaphoreType.DMA,
    ],
)
def _sc_cost_kernel(tokens_hbm, table_hbm, wb_hbm, boffs_hbm, wrot_hbm, out_hbm,
                    idx_v, rows0, rows1, out_v, wb_v, boffs_v, wrot_v,
                    sem0, sem1):
    wid = lax.axis_index("s") * NC + lax.axis_index("c")
    base = wid * TPW

    pltpu.sync_copy(wb_hbm, wb_v)
    pltpu.sync_copy(boffs_hbm, boffs_v)
    pltpu.sync_copy(wrot_hbm, wrot_v)
    pltpu.sync_copy(tokens_hbm.at[pl.ds(base, TPW)], idx_v)

    rows = [rows0, rows1]
    sems = [sem0, sem1]

    def start_gather(c):
        buf = c % 2
        return pltpu.async_copy(
            table_hbm.at[idx_v.at[pl.ds(c * CH, CH)]], rows[buf], sems[buf])

    lane = lax.broadcasted_iota(jnp.int32, (L,), 0)
    bias_vec = plsc.load_gather(wb_v, [jnp.full((L,), D, jnp.int32)])
    G = CH // L

    cp = start_gather(0)
    for c in range(NCH):
        nxt = start_gather(c + 1) if c + 1 < NCH else None
        cp.wait()
        buf = rows[c % 2]

        ridx = tuple(lane + g * L for g in range(G))
        accs0 = (bias_vec,) * G

        def body(i, accs):
            off = i * L
            boff = boffs_v[pl.ds(off, L)]
            wv = wrot_v[pl.ds(off, L)]
            return tuple(
                accs[g] + plsc.load_gather(buf, [ridx[g], boff]) * wv
                for g in range(G))

        accs = lax.fori_loop(0, D, body, accs0)
        for g in range(G):
            out_v[pl.ds(c * CH + g * L, L)] = accs[g]
        cp = nxt

    pltpu.sync_copy(out_v, out_hbm.at[pl.ds(base, TPW)])


def kernel(tokens, cu_seqlens, table, W, b):
    wb = jnp.concatenate([W[:, 0], b, jnp.zeros((7,), jnp.float32)])
    i = jnp.arange(D)[:, None]
    l = jnp.arange(L)[None, :]
    boffs = ((i // L) * L + (l + i % L) % L).astype(jnp.int32)
    wrot = W[:, 0][boffs]
    costs = _sc_cost_kernel(tokens, table, wb, boffs.reshape(-1),
                            wrot.reshape(-1))
    return costs, cu_seqlens


# speedup vs baseline: 3.2541x; 1.3949x over previous
import functools

import jax
import jax.numpy as jnp
from jax import lax
from jax.experimental import pallas as pl
from jax.experimental.pallas import tpu as pltpu
from jax.experimental.pallas import tpu_sc as plsc

TOTAL = 32768
D = 128
NC = 2
NS = 16
L = 16
NW = NC * NS
TPW = TOTAL // NW
CH = 128
NCH = TPW // CH
G = CH // L

_mesh = plsc.VectorSubcoreMesh(core_axis_name="c", subcore_axis_name="s")


@functools.partial(
    pl.kernel,
    mesh=_mesh,
    out_type=jax.ShapeDtypeStruct((TOTAL,), jnp.float32),
    compiler_params=pltpu.CompilerParams(needs_layout_passes=False),
    scratch_types=[
        pltpu.VMEM((TPW,), jnp.int32),
        pltpu.VMEM((CH, D), jnp.float32),
        pltpu.VMEM((CH, D), jnp.float32),
        pltpu.VMEM((TPW,), jnp.float32),
        pltpu.VMEM((D, 1), jnp.float32),
        pltpu.VMEM((1,), jnp.float32),
        pltpu.SemaphoreType.DMA,
        pltpu.SemaphoreType.DMA,
    ],
)
def _sc_cost_kernel(tokens_hbm, table_hbm, w_hbm, b_hbm, out_hbm,
                    idx_v, rows0, rows1, out_v, w_v, b_v, sem0, sem1):
    wid = lax.axis_index("s") * NC + lax.axis_index("c")
    base = wid * TPW

    pltpu.sync_copy(w_hbm, w_v)
    pltpu.sync_copy(b_hbm, b_v)
    pltpu.sync_copy(tokens_hbm.at[pl.ds(base, TPW)], idx_v)

    rows = [rows0, rows1]
    sems = [sem0, sem1]

    def start_gather(c):
        buf = c % 2
        return pltpu.async_copy(
            table_hbm.at[idx_v.at[pl.ds(c * CH, CH)]], rows[buf], sems[buf])

    lane = lax.broadcasted_iota(jnp.int32, (L,), 0)
    zero16 = jnp.zeros((L,), jnp.int32)
    bias_vec = plsc.load_gather(b_v, [zero16])
    ridx = tuple(lane + g * L for g in range(G))

    cp = start_gather(0)
    for c in range(NCH):
        nxt = start_gather(c + 1) if c + 1 < NCH else None
        cp.wait()
        buf = rows[c % 2]
        accs0 = (bias_vec,) * G

        def body(i, accs):
            k = i & (L - 1)
            blk = i - k
            boff = ((lane + k) & (L - 1)) + blk
            wv = plsc.load_gather(w_v, [boff, zero16])
            return tuple(
                accs[g] + plsc.load_gather(buf, [ridx[g], boff]) * wv
                for g in range(G))

        accs = lax.fori_loop(0, D, body, accs0)
        for g in range(G):
            out_v[pl.ds(c * CH + g * L, L)] = accs[g]
        cp = nxt

    pltpu.sync_copy(out_v, out_hbm.at[pl.ds(base, TPW)])


def kernel(tokens, cu_seqlens, table, W, b):
    return _sc_cost_kernel(tokens, table, W, b), cu_seqlens
